# Initial kernel scaffold; baseline (speedup 1.0000x reference)
import jax
import jax.numpy as jnp
from jax.experimental import pallas as pl


def kernel(topk_indices, features):
    raise NotImplementedError("write your pallas kernel here")


# speedup vs baseline: 18.4615x; 18.4615x over previous
import functools

import jax
import jax.numpy as jnp
from jax import lax
from jax.experimental import pallas as pl
from jax.experimental.pallas import tpu as pltpu
from jax.experimental.pallas import tpu_sc as plsc

B, N, K, D = 16, 4096, 20, 64
NC, NS, L = 2, 16, 16
NW = NC * NS
TOTAL_ROWS = B * N * K
IW = 128
G = TOTAL_ROWS // IW
ROWS_PER_W = G // NW
RC = 4
CHUNKS = ROWS_PER_W // RC


def _sc_gather(idx_hbm, feat_hbm, out_hbm, idx_v, rows_v, sem):
    wid = lax.axis_index("s") * NC + lax.axis_index("c")
    off = (wid // 2) * N
    row0 = wid * ROWS_PER_W

    def body(g, _):
        base = row0 + g * RC
        pltpu.sync_copy(idx_hbm.at[pl.ds(base, RC)], idx_v)
        for r in range(RC):
            for j in range(IW // L):
                sl = pl.ds(j * L, L)
                idx_v[r, sl] = idx_v[r, sl] + off
        cps = [
            pltpu.async_copy(feat_hbm.at[idx_v.at[r]], rows_v.at[r], sem)
            for r in range(RC)
        ]
        for cp in cps:
            cp.wait()
        pltpu.sync_copy(rows_v, out_hbm.at[pl.ds(base, RC)])
        return _

    lax.fori_loop(0, CHUNKS, body, 0)


@jax.jit
def kernel(topk_indices, features):
    idx = topk_indices.astype(jnp.int32).reshape(G, IW)
    feat = features.reshape(B * N, D)
    mesh = plsc.VectorSubcoreMesh(core_axis_name="c", subcore_axis_name="s")
    out = pl.kernel(
        _sc_gather,
        out_type=jax.ShapeDtypeStruct((G, IW, D), jnp.float32),
        mesh=mesh,
        scratch_types=[
            pltpu.VMEM((RC, IW), jnp.int32),
            pltpu.VMEM((RC, IW, D), jnp.float32),
            pltpu.SemaphoreType.DMA,
        ],
        compiler_params=pltpu.CompilerParams(use_tc_tiling_on_sc=False),
    )(idx, feat)
    return out.reshape(B, N, K, D)
